# baseline (speedup 1.0000x reference)
import functools

import jax
import jax.numpy as jnp
from jax import lax
from jax.experimental import pallas as pl
from jax.experimental.pallas import tpu as pltpu
from jax.experimental.pallas import tpu_sc as plsc

_POWER = 2
_BATCH, _MAX_LEN = 4096, 50
_EMBED_DIM, _MASK_DIM = 64, 16

_TOTAL = _BATCH * _MAX_LEN
_NW = 32
_PER_W = _TOTAL // _NW
_CHUNK = 128
_NCHUNK = _PER_W // _CHUNK


def _sc_gather(word_table, sidx2d, mask_table, midx2d):
    mesh = plsc.VectorSubcoreMesh(core_axis_name="c", subcore_axis_name="s")

    @functools.partial(
        pl.kernel,
        mesh=mesh,
        out_type=(
            jax.ShapeDtypeStruct((_TOTAL, _EMBED_DIM), jnp.float32),
            jax.ShapeDtypeStruct((_TOTAL, _MASK_DIM), jnp.float32),
        ),
        scratch_types=[
            pltpu.VMEM((_NCHUNK, _CHUNK), jnp.int32),
            pltpu.VMEM((_NCHUNK, _CHUNK), jnp.int32),
            pltpu.VMEM((_CHUNK, _EMBED_DIM), jnp.float32),
            pltpu.VMEM((_CHUNK, _MASK_DIM), jnp.float32),
            pltpu.SemaphoreType.DMA,
            pltpu.SemaphoreType.DMA,
        ],
        compiler_params=pltpu.CompilerParams(use_tc_tiling_on_sc=False),
    )
    def body(word_hbm, sidx_hbm, mtab_hbm, midx_hbm, out_w, out_m,
             sidx_v, midx_v, wrows_v, mrows_v, wsem, msem):
        wid = lax.axis_index("s") * 2 + lax.axis_index("c")
        pltpu.sync_copy(sidx_hbm.at[wid], sidx_v)
        pltpu.sync_copy(midx_hbm.at[wid], midx_v)

        def step(j, carry):
            row0 = (wid * _NCHUNK + j) * _CHUNK
            cw = pltpu.async_copy(word_hbm.at[sidx_v.at[j]], wrows_v, wsem)
            cm = pltpu.async_copy(mtab_hbm.at[midx_v.at[j]], mrows_v, msem)
            cw.wait()
            pltpu.sync_copy(wrows_v, out_w.at[pl.ds(row0, _CHUNK)])
            cm.wait()
            pltpu.sync_copy(mrows_v, out_m.at[pl.ds(row0, _CHUNK)])
            return carry

        lax.fori_loop(0, _NCHUNK, step, 0)

    return body(word_table, sidx2d, mask_table, midx2d)


def _pw_body(mask_ref, pw_ref):
    m = mask_ref[...]
    q = lax.broadcasted_iota(jnp.int32, m.shape, 1)
    left = jnp.min(jnp.where(m == 1, q, jnp.int32(1 << 30)), axis=1,
                   keepdims=True)
    right = left + jnp.sum(m, axis=1, keepdims=True)
    d = jnp.where(q < left, left - q, jnp.where(q > right, q - right, 0))
    num = (100 - d) ** _POWER
    pw_ref[...] = num.astype(jnp.float32) / jnp.float32(100 ** _POWER)


def _position_weight(mask):
    return pl.pallas_call(
        _pw_body,
        out_shape=jax.ShapeDtypeStruct((_BATCH, _MAX_LEN), jnp.float32),
    )(mask)


def kernel(sent, mask, word_table, mask_table):
    sidx2d = sent.reshape(_NW, _NCHUNK, _CHUNK)
    midx2d = mask.reshape(_NW, _NCHUNK, _CHUNK)
    word_rows, mask_rows = _sc_gather(word_table, sidx2d, mask_table, midx2d)
    sent_vec = word_rows.reshape(_BATCH, _MAX_LEN, _EMBED_DIM)
    mask_vec = mask_rows.reshape(_BATCH, _MAX_LEN, _MASK_DIM)
    position_weight = _position_weight(mask)
    return (sent_vec, mask_vec, position_weight)


# speedup vs baseline: 1.8753x; 1.8753x over previous
import functools

import jax
import jax.numpy as jnp
from jax import lax
from jax.experimental import pallas as pl
from jax.experimental.pallas import tpu as pltpu
from jax.experimental.pallas import tpu_sc as plsc

_POWER = 2
_BATCH, _MAX_LEN = 4096, 50
_EMBED_DIM, _MASK_DIM = 64, 16

_TOTAL = _BATCH * _MAX_LEN
_NW = 32
_PER_W = _TOTAL // _NW
_CHUNK = 128
_NCHUNK = _PER_W // _CHUNK
_K = 5
_SUPER = _K * _CHUNK
_NSUPER = _NCHUNK // _K
_MREP = 128


def _sc_gather(word_table, sidx3d, mask_rep, midx3d):
    mesh = plsc.VectorSubcoreMesh(core_axis_name="c", subcore_axis_name="s")

    @functools.partial(
        pl.kernel,
        mesh=mesh,
        out_type=(
            jax.ShapeDtypeStruct((_TOTAL, _EMBED_DIM), jnp.float32),
            jax.ShapeDtypeStruct((_TOTAL, _MASK_DIM), jnp.float32),
        ),
        scratch_types=[
            pltpu.VMEM((_NCHUNK, _CHUNK), jnp.int32),
            pltpu.VMEM((_NCHUNK, _CHUNK), jnp.int32),
            pltpu.VMEM((2, _SUPER, _EMBED_DIM), jnp.float32),
            pltpu.VMEM((2, _SUPER, _MASK_DIM), jnp.float32),
        ] + [pltpu.SemaphoreType.DMA] * 8,
        compiler_params=pltpu.CompilerParams(use_tc_tiling_on_sc=False),
    )
    def body(word_hbm, sidx_hbm, mtab_hbm, midx_hbm, out_w, out_m,
             sidx_v, midx_v, wrows_v, mrows_v,
             wg0, wg1, mg0, mg1, ww0, ww1, mw0, mw1):
        wid = lax.axis_index("s") * 2 + lax.axis_index("c")
        base = wid * _PER_W
        pltpu.sync_copy(sidx_hbm.at[wid], sidx_v)
        pltpu.sync_copy(midx_hbm.at[wid], midx_v)

        wg = (wg0, wg1)
        mg = (mg0, mg1)
        ww = (ww0, ww1)
        mw = (mw0, mw1)

        def fire_gathers(s, b):
            wh, mh = [], []
            for c in range(_K):
                j = s * _K + c
                dst = pl.ds(c * _CHUNK, _CHUNK)
                wh.append(pltpu.async_copy(
                    word_hbm.at[sidx_v.at[j]], wrows_v.at[b].at[dst], wg[b]))
                mh.append(pltpu.async_copy(
                    mtab_hbm.at[midx_v.at[j]], mrows_v.at[b].at[dst], mg[b]))
            return wh + mh

        def fire_writebacks(s, b):
            rows = pl.ds(base + s * _SUPER, _SUPER)
            return [
                pltpu.async_copy(wrows_v.at[b], out_w.at[rows], ww[b]),
                pltpu.async_copy(mrows_v.at[b], out_m.at[rows], mw[b]),
            ]

        gathers = {0: fire_gathers(0, 0)}
        writebacks = {}
        for s in range(_NSUPER):
            b = s % 2
            o = 1 - b
            if s > 0:
                for h in writebacks.pop(s - 1):
                    h.wait()
            if s + 1 < _NSUPER:
                gathers[s + 1] = fire_gathers(s + 1, o)
            for h in gathers.pop(s):
                h.wait()
            writebacks[s] = fire_writebacks(s, b)
        for h in writebacks.pop(_NSUPER - 1):
            h.wait()

    return body(word_table, sidx3d, mask_rep, midx3d)


def _pw_body(mask_ref, pw_ref):
    m = mask_ref[...]
    q = lax.broadcasted_iota(jnp.int32, m.shape, 1)
    left = jnp.min(jnp.where(m == 1, q, jnp.int32(1 << 30)), axis=1,
                   keepdims=True)
    right = left + jnp.sum(m, axis=1, keepdims=True)
    d = jnp.where(q < left, left - q, jnp.where(q > right, q - right, 0))
    num = (100 - d) ** _POWER
    pw_ref[...] = num.astype(jnp.float32) / jnp.float32(100 ** _POWER)


def _position_weight(mask):
    return pl.pallas_call(
        _pw_body,
        out_shape=jax.ShapeDtypeStruct((_BATCH, _MAX_LEN), jnp.float32),
    )(mask)


def kernel(sent, mask, word_table, mask_table):
    sidx3d = sent.reshape(_NW, _NCHUNK, _CHUNK)
    mask_flat = mask.reshape(-1)
    mask_rep = jnp.tile(mask_table, (_MREP // 2, 1))
    rep = (lax.iota(jnp.int32, _TOTAL) & (_MREP // 2 - 1)) << 1
    midx3d = (mask_flat + rep).reshape(_NW, _NCHUNK, _CHUNK)
    word_rows, mask_rows = _sc_gather(word_table, sidx3d, mask_rep, midx3d)
    sent_vec = word_rows.reshape(_BATCH, _MAX_LEN, _EMBED_DIM)
    mask_vec = mask_rows.reshape(_BATCH, _MAX_LEN, _MASK_DIM)
    position_weight = _position_weight(mask)
    return (sent_vec, mask_vec, position_weight)


# speedup vs baseline: 2.0791x; 1.1087x over previous
import functools

import jax
import jax.numpy as jnp
from jax import lax
from jax.experimental import pallas as pl
from jax.experimental.pallas import tpu as pltpu
from jax.experimental.pallas import tpu_sc as plsc

_POWER = 2
_BATCH, _MAX_LEN = 4096, 50
_VOCAB = 1000000
_EMBED_DIM, _MASK_DIM = 64, 16

_NW = 32
_BPW = _BATCH // _NW
_NUNIT = _MAX_LEN


def _sc_gather(wt_packed, sent_t):
    mesh = plsc.VectorSubcoreMesh(core_axis_name="c", subcore_axis_name="s")

    @functools.partial(
        pl.kernel,
        mesh=mesh,
        out_type=jax.ShapeDtypeStruct((_MAX_LEN, _EMBED_DIM, _BATCH),
                                      jnp.float32),
        scratch_types=[
            pltpu.VMEM((_NUNIT, _BPW), jnp.int32),
            pltpu.VMEM((_NUNIT, _BPW), jnp.int32),
            pltpu.VMEM((_BPW, 128), jnp.float32),
            pltpu.VMEM((_BPW, 128), jnp.float32),
            pltpu.VMEM((_EMBED_DIM, _BPW), jnp.float32),
            pltpu.VMEM((_EMBED_DIM, _BPW), jnp.float32),
        ] + [pltpu.SemaphoreType.DMA] * 4,
        compiler_params=pltpu.CompilerParams(use_tc_tiling_on_sc=True,
                                             needs_layout_passes=False),
    )
    def body(wt_hbm, sidx_hbm, out_hbm,
             idx_v, kidx_v, gbuf0, gbuf1, tbuf0, tbuf1,
             g0, g1, w0, w1):
        wid = lax.axis_index("s") * 2 + lax.axis_index("c")
        b0 = wid * _BPW
        pltpu.sync_copy(sidx_hbm.at[:, pl.ds(b0, _BPW)], idx_v)

        def shift_row(j, carry):
            for c in range(_BPW // 16):
                v = idx_v[j, pl.ds(c * 16, 16)]
                kidx_v[j, pl.ds(c * 16, 16)] = v >> 1
            return carry
        lax.fori_loop(0, _NUNIT, shift_row, 0)

        gbuf = (gbuf0, gbuf1)
        tbuf = (tbuf0, tbuf1)
        gsem = (g0, g1)
        wsem = (w0, w1)

        def start_gather(j, b):
            return pltpu.async_copy(wt_hbm.at[kidx_v.at[j]], gbuf[b], gsem[b])

        def start_writeback(j, b):
            return pltpu.async_copy(
                tbuf[b], out_hbm.at[j, :, pl.ds(b0, _BPW)], wsem[b])

        def wait_gather(j, b):
            pltpu.make_async_copy(wt_hbm.at[kidx_v.at[j]], gbuf[b],
                                  gsem[b]).wait()

        def wait_writeback(j, b):
            pltpu.make_async_copy(
                tbuf[b], out_hbm.at[j, :, pl.ds(b0, _BPW)], wsem[b]).wait()

        def transpose_unit(j, b):
            iota = lax.iota(jnp.int32, 16)
            for c in range(_BPW // 16):
                rows = iota + (c * 16)
                h6 = (idx_v[j, pl.ds(c * 16, 16)] & 1) << 6

                def dloop(d, carry):
                    v = plsc.load_gather(gbuf[b], [rows, h6 + d])
                    tbuf[b][d, pl.ds(c * 16, 16)] = v
                    return carry
                lax.fori_loop(0, _EMBED_DIM, dloop, 0)

        h = start_gather(0, 0)
        h = start_gather(1, 1)
        del h

        def unit_pair(s, carry):
            for b in (0, 1):
                j = 2 * s + b

                @pl.when(j >= 2)
                def _():
                    wait_writeback(j - 2, b)

                wait_gather(j, b)
                transpose_unit(j, b)

                @pl.when(j < _NUNIT - 2)
                def _():
                    start_gather(j + 2, b)

                start_writeback(j, b)
            return carry

        lax.fori_loop(0, _NUNIT // 2, unit_pair, 0)
        wait_writeback(_NUNIT - 2, 0)
        wait_writeback(_NUNIT - 1, 1)

    return body(wt_packed, sent_t)


def _tc_body(mask_t_ref, mtab_t_ref, m3_ref, pw_ref):
    m = mask_t_ref[...]
    q = lax.broadcasted_iota(jnp.int32, m.shape, 0)
    left = jnp.min(jnp.where(m == 1, q, jnp.int32(1 << 30)), axis=0,
                   keepdims=True)
    right = left + jnp.sum(m, axis=0, keepdims=True)
    d = jnp.where(q < left, left - q, jnp.where(q > right, q - right, 0))
    num = (100 - d) ** _POWER
    pw_ref[...] = num.astype(jnp.float32) / jnp.float32(100 ** _POWER)

    mt = mtab_t_ref[...]
    mt0 = mt[:, 0].reshape(1, _MASK_DIM, 1)
    mt1 = mt[:, 1].reshape(1, _MASK_DIM, 1)
    sel = (m == 1)[:, None, :]
    m3_ref[...] = jnp.where(sel, mt1, mt0)


def _tc_mask_pw(mask_t, mtab_t):
    grid = 8
    bb = _BATCH // grid
    return pl.pallas_call(
        _tc_body,
        grid=(grid,),
        in_specs=[
            pl.BlockSpec((_MAX_LEN, bb), lambda i: (0, i)),
            pl.BlockSpec((_MASK_DIM, 2), lambda i: (0, 0)),
        ],
        out_specs=[
            pl.BlockSpec((_MAX_LEN, _MASK_DIM, bb), lambda i: (0, 0, i)),
            pl.BlockSpec((_MAX_LEN, bb), lambda i: (0, i)),
        ],
        out_shape=[
            jax.ShapeDtypeStruct((_MAX_LEN, _MASK_DIM, _BATCH), jnp.float32),
            jax.ShapeDtypeStruct((_MAX_LEN, _BATCH), jnp.float32),
        ],
    )(mask_t, mtab_t)


def kernel(sent, mask, word_table, mask_table):
    wt_packed = word_table.reshape(_VOCAB // 2, 2 * _EMBED_DIM)
    sent_t = sent.T
    mask_t = mask.T
    mtab_t = mask_table.T
    w3 = _sc_gather(wt_packed, sent_t)
    m3, pw_t = _tc_mask_pw(mask_t, mtab_t)
    sent_vec = jnp.transpose(w3, (2, 0, 1))
    mask_vec = jnp.transpose(m3, (2, 0, 1))
    position_weight = pw_t.T
    return (sent_vec, mask_vec, position_weight)


# speedup vs baseline: 2.9405x; 1.4143x over previous
import functools

import jax
import jax.numpy as jnp
from jax import lax
from jax.experimental import pallas as pl
from jax.experimental.pallas import tpu as pltpu
from jax.experimental.pallas import tpu_sc as plsc

_POWER = 2
_BATCH, _MAX_LEN = 4096, 50
_VOCAB = 1000000
_EMBED_DIM, _MASK_DIM = 64, 16

_NW = 32
_BPW = _BATCH // _NW
_NUNIT = _MAX_LEN
_PACK_N = 4096
_PACK_GRID = 245
_PACK_ROWS = _PACK_GRID * (_PACK_N // 2)


def _pack_body(x_ref, o_ref):
    x = x_ref[...]
    xt = jnp.transpose(x, (1, 0))
    o_ref[...] = jnp.concatenate([xt[: _PACK_N // 2], xt[_PACK_N // 2:]],
                                 axis=1)


def _tc_pack(word_table):
    wtT = word_table.T
    return pl.pallas_call(
        _pack_body,
        grid=(_PACK_GRID,),
        in_specs=[pl.BlockSpec((64, _PACK_N), lambda i: (0, i))],
        out_specs=pl.BlockSpec((_PACK_N // 2, 128), lambda i: (i, 0)),
        out_shape=jax.ShapeDtypeStruct((_PACK_ROWS, 128), jnp.float32),
    )(wtT)


def _sc_gather(wt_packed, sent_t):
    mesh = plsc.VectorSubcoreMesh(core_axis_name="c", subcore_axis_name="s")

    @functools.partial(
        pl.kernel,
        mesh=mesh,
        out_type=jax.ShapeDtypeStruct((_MAX_LEN, _EMBED_DIM, _BATCH),
                                      jnp.float32),
        scratch_types=[
            pltpu.VMEM((_NUNIT, _BPW), jnp.int32),
            pltpu.VMEM((_NUNIT, _BPW), jnp.int32),
            pltpu.VMEM((_BPW, 128), jnp.float32),
            pltpu.VMEM((_BPW, 128), jnp.float32),
            pltpu.VMEM((_EMBED_DIM, _BPW), jnp.float32),
            pltpu.VMEM((_EMBED_DIM, _BPW), jnp.float32),
        ] + [pltpu.SemaphoreType.DMA] * 4,
        compiler_params=pltpu.CompilerParams(use_tc_tiling_on_sc=True,
                                             needs_layout_passes=False),
    )
    def body(wt_hbm, sidx_hbm, out_hbm,
             idx_v, kidx_v, gbuf0, gbuf1, tbuf0, tbuf1,
             g0, g1, w0, w1):
        wid = lax.axis_index("s") * 2 + lax.axis_index("c")
        b0 = wid * _BPW
        pltpu.sync_copy(sidx_hbm.at[:, pl.ds(b0, _BPW)], idx_v)

        def shift_row(j, carry):
            for c in range(_BPW // 16):
                v = idx_v[j, pl.ds(c * 16, 16)]
                kidx_v[j, pl.ds(c * 16, 16)] = ((v >> 12) << 11) | (v & 2047)
            return carry
        lax.fori_loop(0, _NUNIT, shift_row, 0)

        gbuf = (gbuf0, gbuf1)
        tbuf = (tbuf0, tbuf1)
        gsem = (g0, g1)
        wsem = (w0, w1)

        def start_gather(j, b):
            return pltpu.async_copy(wt_hbm.at[kidx_v.at[j]], gbuf[b], gsem[b])

        def start_writeback(j, b):
            return pltpu.async_copy(
                tbuf[b], out_hbm.at[j, :, pl.ds(b0, _BPW)], wsem[b])

        def wait_gather(j, b):
            pltpu.make_async_copy(wt_hbm.at[kidx_v.at[j]], gbuf[b],
                                  gsem[b]).wait()

        def wait_writeback(j, b):
            pltpu.make_async_copy(
                tbuf[b], out_hbm.at[j, :, pl.ds(b0, _BPW)], wsem[b]).wait()

        def transpose_unit(j, b):
            iota = lax.iota(jnp.int32, 16)
            for c in range(_BPW // 16):
                rows = iota + (c * 16)
                h6 = ((idx_v[j, pl.ds(c * 16, 16)] >> 11) & 1) << 6

                def dloop(d8, carry):
                    d0 = d8 * 8
                    for dd in range(8):
                        v = plsc.load_gather(gbuf[b], [rows, h6 + (d0 + dd)])
                        tbuf[b][d0 + dd, pl.ds(c * 16, 16)] = v
                    return carry
                lax.fori_loop(0, _EMBED_DIM // 8, dloop, 0)

        h = start_gather(0, 0)
        h = start_gather(1, 1)
        del h

        def unit_pair(s, carry):
            for b in (0, 1):
                j = 2 * s + b

                @pl.when(j >= 2)
                def _():
                    wait_writeback(j - 2, b)

                wait_gather(j, b)
                transpose_unit(j, b)

                @pl.when(j < _NUNIT - 2)
                def _():
                    start_gather(j + 2, b)

                start_writeback(j, b)
            return carry

        lax.fori_loop(0, _NUNIT // 2, unit_pair, 0)
        wait_writeback(_NUNIT - 2, 0)
        wait_writeback(_NUNIT - 1, 1)

    return body(wt_packed, sent_t)


def _tc_body(mask_t_ref, mtab_t_ref, m3_ref, pw_ref):
    m = mask_t_ref[...]
    q = lax.broadcasted_iota(jnp.int32, m.shape, 0)
    left = jnp.min(jnp.where(m == 1, q, jnp.int32(1 << 30)), axis=0,
                   keepdims=True)
    right = left + jnp.sum(m, axis=0, keepdims=True)
    d = jnp.where(q < left, left - q, jnp.where(q > right, q - right, 0))
    num = (100 - d) ** _POWER
    pw_ref[...] = num.astype(jnp.float32) / jnp.float32(100 ** _POWER)

    mt = mtab_t_ref[...]
    mt0 = mt[:, 0].reshape(1, _MASK_DIM, 1)
    mt1 = mt[:, 1].reshape(1, _MASK_DIM, 1)
    sel = (m == 1)[:, None, :]
    m3_ref[...] = jnp.where(sel, mt1, mt0)


def _tc_mask_pw(mask_t, mtab_t):
    grid = 8
    bb = _BATCH // grid
    return pl.pallas_call(
        _tc_body,
        grid=(grid,),
        in_specs=[
            pl.BlockSpec((_MAX_LEN, bb), lambda i: (0, i)),
            pl.BlockSpec((_MASK_DIM, 2), lambda i: (0, 0)),
        ],
        out_specs=[
            pl.BlockSpec((_MAX_LEN, _MASK_DIM, bb), lambda i: (0, 0, i)),
            pl.BlockSpec((_MAX_LEN, bb), lambda i: (0, i)),
        ],
        out_shape=[
            jax.ShapeDtypeStruct((_MAX_LEN, _MASK_DIM, _BATCH), jnp.float32),
            jax.ShapeDtypeStruct((_MAX_LEN, _BATCH), jnp.float32),
        ],
    )(mask_t, mtab_t)


def kernel(sent, mask, word_table, mask_table):
    wt_packed = _tc_pack(word_table)
    sent_t = sent.T
    mask_t = mask.T
    mtab_t = mask_table.T
    w3 = _sc_gather(wt_packed, sent_t)
    m3, pw_t = _tc_mask_pw(mask_t, mtab_t)
    sent_vec = jnp.transpose(w3, (2, 0, 1))
    mask_vec = jnp.transpose(m3, (2, 0, 1))
    position_weight = pw_t.T
    return (sent_vec, mask_vec, position_weight)
